# baseline (speedup 1.0000x reference)
import jax
import jax.numpy as jnp
from jax.experimental import pallas as pl

N = 8192
D = 64
BM = 256


def _matmul_block(a_ref, x_ref, o_ref):
    o_ref[...] = jnp.dot(a_ref[...], x_ref[...],
                         preferred_element_type=jnp.float32)


def kernel(x, A_hat):
    return pl.pallas_call(
        _matmul_block,
        grid=(N // BM,),
        in_specs=[
            pl.BlockSpec((BM, N), lambda i: (i, 0)),
            pl.BlockSpec((N, D), lambda i: (0, 0)),
        ],
        out_specs=pl.BlockSpec((BM, D), lambda i: (i, 0)),
        out_shape=jax.ShapeDtypeStruct((N, D), jnp.float32),
    )(A_hat, x)
